# baseline (speedup 1.0000x reference)
import jax
import jax.numpy as jnp
from jax import lax
from jax.experimental import pallas as pl
from jax.experimental.pallas import tpu as pltpu

_DIM = 8
_CB = 512
_NQ = 16
_NR = 32
_LOGVAR_MIN = -30.0
_LOGVAR_MAX = 20.0
_KL_SCALE = 1.4426 * 0.5


def _body(z_ref, w_ref, p_ref, zhat_ref, idx_ref, kl_ref):
    zb = z_ref[0]
    mu = zb[0:16, :]
    lv = zb[16:32, :]
    lvc = jnp.clip(lv, _LOGVAR_MIN, _LOGVAR_MAX)
    var = jnp.exp(lvc)
    inv = 1.0 / var
    acts = jnp.concatenate([0.5 - 0.5 * inv, mu * inv], axis=0)
    a_hi = acts.astype(jnp.bfloat16)
    rem1 = acts - a_hi.astype(jnp.float32)
    a_mid = rem1.astype(jnp.bfloat16)
    a_lo = (rem1 - a_mid.astype(jnp.float32)).astype(jnp.bfloat16)
    a_big = jnp.concatenate(
        [a_hi, a_hi, a_mid, a_mid, a_hi, a_lo], axis=0)
    score = lax.dot_general(
        w_ref[...], a_big, (((1,), (0,)), ((), ())),
        preferred_element_type=jnp.float32)
    blk = score.shape[1]
    iota = lax.broadcasted_iota(jnp.int32, (_CB, blk), 0)
    s0 = score[0:_CB, :]
    s1 = score[_CB:2 * _CB, :]
    m0 = jnp.max(s0, axis=0, keepdims=True)
    m1 = jnp.max(s1, axis=0, keepdims=True)
    idx0 = jnp.min(jnp.where(s0 == m0, iota, _CB), axis=0)
    idx1 = jnp.min(jnp.where(s1 == m1, iota, _CB), axis=0)
    idx_ref[0, 0, :] = idx0
    idx_ref[0, 1, :] = idx1

    iota_r = lax.broadcasted_iota(jnp.int32, (_NR, blk), 0)
    ohr0 = (iota_r == (idx0 & (_NR - 1))[None, :]).astype(jnp.bfloat16)
    ohr1 = (iota_r == (idx1 & (_NR - 1))[None, :]).astype(jnp.bfloat16)
    ohfull = jnp.concatenate([ohr0, ohr1, ohr0, ohr1], axis=0)
    u_int = lax.dot_general(
        p_ref[...], ohfull, (((1,), (0,)), ((), ())),
        preferred_element_type=jnp.float32)
    parity = lax.broadcasted_iota(jnp.int32, (16, blk), 0) & 1
    q0_b = jnp.broadcast_to((idx0 >> 5)[None, :], (16, blk))
    q1_b = jnp.broadcast_to((idx1 >> 5)[None, :], (16, blk))
    q_int = jnp.where(parity == 1, q1_b, q0_b)
    lvl = [u_int[qq * 16:(qq + 1) * 16, :] for qq in range(_NQ)]
    bit = 1
    while len(lvl) > 1:
        pick = (q_int & bit) != 0
        lvl = [jnp.where(pick, hi, lo) for lo, hi in zip(lvl[::2], lvl[1::2])]
        bit <<= 1
    zhat_ref[0] = lvl[0]

    part = jnp.sum(mu * mu + var - 1.0 - lvc)

    @pl.when(jnp.logical_and(pl.program_id(0) == 0, pl.program_id(1) == 0))
    def _init():
        kl_ref[...] = jnp.zeros_like(kl_ref)

    kl_ref[...] += part


def _codebook_mats(prior_samples):
    ps = prior_samples.astype(jnp.float32)
    ps2 = ps * ps
    j = jnp.arange(16)
    dsel = j // 2
    par = j % 2
    gsel = jnp.arange(2)[:, None, None]
    wa = jnp.where(par[None, None, :] == gsel, ps2[:, dsel][None], 0.0)
    wb = jnp.where(par[None, None, :] == gsel, ps[:, dsel][None], 0.0)
    w_int = jnp.concatenate(
        [wa.reshape(2 * _CB, 16), wb.reshape(2 * _CB, 16)], axis=1)
    w_hi = w_int.astype(jnp.bfloat16)
    wrem = w_int - w_hi.astype(jnp.float32)
    w_mid = wrem.astype(jnp.bfloat16)
    w_lo = (wrem - w_mid.astype(jnp.float32)).astype(jnp.bfloat16)
    w_big = jnp.concatenate(
        [w_hi, w_mid, w_hi, w_mid, w_lo, w_hi], axis=1)

    arr = ps.reshape(_NQ, _NR, _DIM)
    pq = arr[:, :, dsel]
    parr = (jnp.arange(2)[None, :, None, None] == par[None, None, None, :])
    p2 = jnp.where(parr, pq[:, None, :, :], 0.0)
    p_base = p2.transpose(0, 3, 1, 2).reshape(_NQ * 16, 2 * _NR)
    p_hi = p_base.astype(jnp.bfloat16)
    p_lo = (p_base - p_hi.astype(jnp.float32)).astype(jnp.bfloat16)
    p_full = jnp.concatenate([p_hi, p_lo], axis=1)
    return w_big, p_full


def kernel(z, prior_samples):
    batch, chans, hh, ww = z.shape
    spatial = hh * ww
    blk = 4096
    zr = z.reshape(batch, chans, spatial)
    w_big, p_full = _codebook_mats(prior_samples)
    grid = (batch, spatial // blk)
    zhat3, idx3, klsum = pl.pallas_call(
        _body,
        grid=grid,
        in_specs=[
            pl.BlockSpec((1, chans, blk), lambda b, s: (b, 0, s)),
            pl.BlockSpec((2 * _CB, 192), lambda b, s: (0, 0)),
            pl.BlockSpec((_NQ * 16, 128), lambda b, s: (0, 0)),
        ],
        out_specs=[
            pl.BlockSpec((1, 16, blk), lambda b, s: (b, 0, s)),
            pl.BlockSpec((1, 2, blk), lambda b, s: (b, 0, s)),
            pl.BlockSpec((1, 1), lambda b, s: (0, 0)),
        ],
        out_shape=[
            jax.ShapeDtypeStruct((batch, 16, spatial), jnp.float32),
            jax.ShapeDtypeStruct((batch, 2, spatial), jnp.int32),
            jax.ShapeDtypeStruct((1, 1), jnp.float32),
        ],
    )(zr, w_big, p_full)
    zhat = zhat3.reshape(batch, 16, hh, ww)
    indices = idx3.reshape(batch, 2, hh, ww)
    kl_loss = klsum[0, 0] * jnp.float32(_KL_SCALE / (batch * spatial * 2))
    return (zhat, kl_loss, indices)


import functools
from jax.experimental.pallas import tpu_sc as plsc


def _body_nogather(z_ref, w_ref, idx_ref, kl_ref):
    zb = z_ref[0]
    mu = zb[0:16, :]
    lv = zb[16:32, :]
    lvc = jnp.clip(lv, _LOGVAR_MIN, _LOGVAR_MAX)
    var = jnp.exp(lvc)
    inv = 1.0 / var
    acts = jnp.concatenate([0.5 - 0.5 * inv, mu * inv], axis=0)
    a_hi = acts.astype(jnp.bfloat16)
    rem1 = acts - a_hi.astype(jnp.float32)
    a_mid = rem1.astype(jnp.bfloat16)
    a_lo = (rem1 - a_mid.astype(jnp.float32)).astype(jnp.bfloat16)
    a_big = jnp.concatenate([a_hi, a_hi, a_mid, a_mid, a_hi, a_lo], axis=0)
    score = lax.dot_general(
        w_ref[...], a_big, (((1,), (0,)), ((), ())),
        preferred_element_type=jnp.float32)
    blk = score.shape[1]
    iota = lax.broadcasted_iota(jnp.int32, (_CB, blk), 0)
    s0 = score[0:_CB, :]
    s1 = score[_CB:2 * _CB, :]
    m0 = jnp.max(s0, axis=0, keepdims=True)
    m1 = jnp.max(s1, axis=0, keepdims=True)
    idx_ref[0, 0, :] = jnp.min(jnp.where(s0 == m0, iota, _CB), axis=0)
    idx_ref[0, 1, :] = jnp.min(jnp.where(s1 == m1, iota, _CB), axis=0)
    part = jnp.sum(mu * mu + var - 1.0 - lvc)

    @pl.when(jnp.logical_and(pl.program_id(0) == 0, pl.program_id(1) == 0))
    def _init():
        kl_ref[...] = jnp.zeros_like(kl_ref)

    kl_ref[...] += part


def _make_sc_gather(n_tokens):
    info = plsc.get_sparse_core_info()
    nw = info.num_cores * info.num_subcores
    per_w = n_tokens // nw
    n_el = per_w * _DIM
    mesh = plsc.VectorSubcoreMesh(core_axis_name="c", subcore_axis_name="s")

    @functools.partial(
        pl.kernel, mesh=mesh,
        compiler_params=pltpu.CompilerParams(needs_layout_passes=False),
        out_type=[jax.ShapeDtypeStruct((n_tokens * _DIM,), jnp.float32),
                  jax.ShapeDtypeStruct((n_tokens * _DIM,), jnp.float32)],
        scratch_types=[
            pltpu.VMEM((_CB * _DIM,), jnp.float32),
            pltpu.VMEM((per_w,), jnp.int32),
            pltpu.VMEM((per_w,), jnp.int32),
            pltpu.VMEM((n_el,), jnp.float32),
            pltpu.VMEM((n_el,), jnp.float32),
        ],
    )
    def sc_gather(table_hbm, i0_hbm, i1_hbm, o0_hbm, o1_hbm,
                  tab_v, i0_v, i1_v, r0_v, r1_v):
        wid = lax.axis_index("s") * info.num_cores + lax.axis_index("c")
        base = wid * per_w
        pltpu.sync_copy(table_hbm, tab_v)
        pltpu.sync_copy(i0_hbm.at[pl.ds(base, per_w)], i0_v)
        pltpu.sync_copy(i1_hbm.at[pl.ds(base, per_w)], i1_v)
        lanes = lax.iota(jnp.int32, 16)

        def step(v, _):
            p0 = v * 16
            pos = p0 + lanes
            tvec = pos >> 3
            dvec = pos & 7
            t0 = plsc.load_gather(i0_v, [tvec])
            r0_v[pl.ds(p0, 16)] = plsc.load_gather(tab_v, [t0 * _DIM + dvec])
            t1 = plsc.load_gather(i1_v, [tvec])
            r1_v[pl.ds(p0, 16)] = plsc.load_gather(tab_v, [t1 * _DIM + dvec])
            return _

        lax.fori_loop(0, n_el // 16, step, 0, unroll=8)
        pltpu.sync_copy(r0_v, o0_hbm.at[pl.ds(base * _DIM, n_el)])
        pltpu.sync_copy(r1_v, o1_hbm.at[pl.ds(base * _DIM, n_el)])

    return sc_gather


def _kernel_sc_variant(z, prior_samples):
    batch, chans, hh, ww = z.shape
    spatial = hh * ww
    blk = 4096
    zr = z.reshape(batch, chans, spatial)
    w_big, _ = _codebook_mats(prior_samples)
    grid = (batch, spatial // blk)
    idx3, klsum = pl.pallas_call(
        _body_nogather,
        grid=grid,
        in_specs=[
            pl.BlockSpec((1, chans, blk), lambda b, s: (b, 0, s)),
            pl.BlockSpec((2 * _CB, 192), lambda b, s: (0, 0)),
        ],
        out_specs=[
            pl.BlockSpec((1, 2, blk), lambda b, s: (b, 0, s)),
            pl.BlockSpec((1, 1), lambda b, s: (0, 0)),
        ],
        out_shape=[
            jax.ShapeDtypeStruct((batch, 2, spatial), jnp.int32),
            jax.ShapeDtypeStruct((1, 1), jnp.float32),
        ],
    )(zr, w_big)
    n_tokens = batch * spatial
    i0 = idx3[:, 0, :].reshape(n_tokens)
    i1 = idx3[:, 1, :].reshape(n_tokens)
    table = prior_samples.astype(jnp.float32).reshape(_CB * _DIM)
    o0, o1 = _make_sc_gather(n_tokens)(table, i0, i1)
    z0 = o0.reshape(batch, spatial, _DIM)
    z1 = o1.reshape(batch, spatial, _DIM)
    zh = jnp.stack([z0, z1], axis=-1).reshape(batch, spatial, 16)
    zhat = zh.transpose(0, 2, 1).reshape(batch, 16, hh, ww)
    indices = idx3.reshape(batch, 2, hh, ww)
    kl_loss = klsum[0, 0] * jnp.float32(_KL_SCALE / (batch * spatial * 2))
    return (zhat, kl_loss, indices)


kernel = _kernel_sc_variant


# speedup vs baseline: 2.3403x; 2.3403x over previous
import jax
import jax.numpy as jnp
from jax import lax
from jax.experimental import pallas as pl
from jax.experimental.pallas import tpu as pltpu

_DIM = 8
_CB = 512
_NQ = 16
_NR = 32
_LOGVAR_MIN = -30.0
_LOGVAR_MAX = 20.0
_KL_SCALE = 1.4426 * 0.5


def _body(z_ref, w_ref, p_ref, zhat_ref, idx_ref, kl_ref):
    zb = z_ref[0]
    mu = zb[0:16, :]
    lv = zb[16:32, :]
    lvc = jnp.clip(lv, _LOGVAR_MIN, _LOGVAR_MAX)
    var = jnp.exp(lvc)
    std = jnp.exp(0.5 * lvc)
    w2 = 1.0 / (2.0 * (std * std))
    acts = jnp.concatenate([0.5 - w2, (mu + mu) * w2], axis=0)
    a_hi = acts.astype(jnp.bfloat16)
    rem1 = acts - a_hi.astype(jnp.float32)
    a_mid = rem1.astype(jnp.bfloat16)
    a_lo = (rem1 - a_mid.astype(jnp.float32)).astype(jnp.bfloat16)
    a_big = jnp.concatenate(
        [a_hi, a_hi, a_mid, a_mid, a_hi, a_lo], axis=0)
    score = lax.dot_general(
        w_ref[...], a_big, (((1,), (0,)), ((), ())),
        preferred_element_type=jnp.float32)
    blk = score.shape[1]
    iota = lax.broadcasted_iota(jnp.int32, (_CB, blk), 0)
    s0 = score[0:_CB, :]
    s1 = score[_CB:2 * _CB, :]
    m0 = jnp.max(s0, axis=0, keepdims=True)
    m1 = jnp.max(s1, axis=0, keepdims=True)
    idx0 = jnp.min(jnp.where(s0 == m0, iota, _CB), axis=0)
    idx1 = jnp.min(jnp.where(s1 == m1, iota, _CB), axis=0)
    idx_ref[0, 0, :] = idx0
    idx_ref[0, 1, :] = idx1

    iota_r = lax.broadcasted_iota(jnp.int32, (_NR, blk), 0)
    ohr0 = (iota_r == (idx0 & (_NR - 1))[None, :]).astype(jnp.bfloat16)
    ohr1 = (iota_r == (idx1 & (_NR - 1))[None, :]).astype(jnp.bfloat16)
    ohfull = jnp.concatenate([ohr0, ohr1, ohr0, ohr1], axis=0)
    u_int = lax.dot_general(
        p_ref[...], ohfull, (((1,), (0,)), ((), ())),
        preferred_element_type=jnp.float32)
    parity = lax.broadcasted_iota(jnp.int32, (16, blk), 0) & 1
    q0_b = jnp.broadcast_to((idx0 >> 5)[None, :], (16, blk))
    q1_b = jnp.broadcast_to((idx1 >> 5)[None, :], (16, blk))
    q_int = jnp.where(parity == 1, q1_b, q0_b)
    lvl = [u_int[qq * 16:(qq + 1) * 16, :] for qq in range(_NQ)]
    bit = 1
    while len(lvl) > 1:
        pick = (q_int & bit) != 0
        lvl = [jnp.where(pick, hi, lo) for lo, hi in zip(lvl[::2], lvl[1::2])]
        bit <<= 1
    zhat_ref[0] = lvl[0]

    part = jnp.sum(mu * mu + var - 1.0 - lvc)

    @pl.when(jnp.logical_and(pl.program_id(0) == 0, pl.program_id(1) == 0))
    def _init():
        kl_ref[...] = jnp.zeros_like(kl_ref)

    kl_ref[...] += part


def _codebook_mats(prior_samples):
    ps = prior_samples.astype(jnp.float32)
    ps2 = ps * ps
    j = jnp.arange(16)
    dsel = j // 2
    par = j % 2
    gsel = jnp.arange(2)[:, None, None]
    wa = jnp.where(par[None, None, :] == gsel, ps2[:, dsel][None], 0.0)
    wb = jnp.where(par[None, None, :] == gsel, ps[:, dsel][None], 0.0)
    w_int = jnp.concatenate(
        [wa.reshape(2 * _CB, 16), wb.reshape(2 * _CB, 16)], axis=1)
    w_hi = w_int.astype(jnp.bfloat16)
    wrem = w_int - w_hi.astype(jnp.float32)
    w_mid = wrem.astype(jnp.bfloat16)
    w_lo = (wrem - w_mid.astype(jnp.float32)).astype(jnp.bfloat16)
    w_big = jnp.concatenate(
        [w_hi, w_mid, w_hi, w_mid, w_lo, w_hi], axis=1)

    arr = ps.reshape(_NQ, _NR, _DIM)
    pq = arr[:, :, dsel]
    parr = (jnp.arange(2)[None, :, None, None] == par[None, None, None, :])
    p2 = jnp.where(parr, pq[:, None, :, :], 0.0)
    p_base = p2.transpose(0, 3, 1, 2).reshape(_NQ * 16, 2 * _NR)
    p_hi = p_base.astype(jnp.bfloat16)
    p_lo = (p_base - p_hi.astype(jnp.float32)).astype(jnp.bfloat16)
    p_full = jnp.concatenate([p_hi, p_lo], axis=1)
    return w_big, p_full


def kernel(z, prior_samples):
    batch, chans, hh, ww = z.shape
    spatial = hh * ww
    blk = 4096
    zr = z.reshape(batch, chans, spatial)
    w_big, p_full = _codebook_mats(prior_samples)
    grid = (batch, spatial // blk)
    zhat3, idx3, klsum = pl.pallas_call(
        _body,
        grid=grid,
        in_specs=[
            pl.BlockSpec((1, chans, blk), lambda b, s: (b, 0, s)),
            pl.BlockSpec((2 * _CB, 192), lambda b, s: (0, 0)),
            pl.BlockSpec((_NQ * 16, 128), lambda b, s: (0, 0)),
        ],
        out_specs=[
            pl.BlockSpec((1, 16, blk), lambda b, s: (b, 0, s)),
            pl.BlockSpec((1, 2, blk), lambda b, s: (b, 0, s)),
            pl.BlockSpec((1, 1), lambda b, s: (0, 0)),
        ],
        out_shape=[
            jax.ShapeDtypeStruct((batch, 16, spatial), jnp.float32),
            jax.ShapeDtypeStruct((batch, 2, spatial), jnp.int32),
            jax.ShapeDtypeStruct((1, 1), jnp.float32),
        ],
    )(zr, w_big, p_full)
    zhat = zhat3.reshape(batch, 16, hh, ww)
    indices = idx3.reshape(batch, 2, hh, ww)
    kl_loss = klsum[0, 0] * jnp.float32(_KL_SCALE / (batch * spatial * 2))
    return (zhat, kl_loss, indices)


# speedup vs baseline: 2.3985x; 1.0248x over previous
import jax
import jax.numpy as jnp
from jax import lax
from jax.experimental import pallas as pl
from jax.experimental.pallas import tpu as pltpu

_DIM = 8
_CB = 512
_NQ = 16
_NR = 32
_LOGVAR_MIN = -30.0
_LOGVAR_MAX = 20.0
_KL_SCALE = 1.4426 * 0.5


def _body(z_ref, w_ref, p_ref, zhat_ref, idx_ref, kl_ref):
    zb = z_ref[0]
    mu = zb[0:16, :]
    lv = zb[16:32, :]
    lvc = jnp.clip(lv, _LOGVAR_MIN, _LOGVAR_MAX)
    var = jnp.exp(lvc)
    inv = 1.0 / var
    acts = jnp.concatenate([0.5 - 0.5 * inv, mu * inv], axis=0)
    a_hi = acts.astype(jnp.bfloat16)
    rem1 = acts - a_hi.astype(jnp.float32)
    a_mid = rem1.astype(jnp.bfloat16)
    a_lo = (rem1 - a_mid.astype(jnp.float32)).astype(jnp.bfloat16)
    a_big = jnp.concatenate(
        [a_hi, a_hi, a_mid, a_mid, a_hi, a_lo], axis=0)
    score = lax.dot_general(
        w_ref[...], a_big, (((1,), (0,)), ((), ())),
        preferred_element_type=jnp.float32)
    blk = score.shape[1]
    iota = lax.broadcasted_iota(jnp.int32, (_CB, blk), 0)
    s0 = score[0:_CB, :]
    s1 = score[_CB:2 * _CB, :]
    m0 = jnp.max(s0, axis=0, keepdims=True)
    m1 = jnp.max(s1, axis=0, keepdims=True)
    idx0 = jnp.min(jnp.where(s0 == m0, iota, _CB), axis=0)
    idx1 = jnp.min(jnp.where(s1 == m1, iota, _CB), axis=0)
    idx_ref[0, 0, :] = idx0
    idx_ref[0, 1, :] = idx1

    iota_r = lax.broadcasted_iota(jnp.int32, (_NR, blk), 0)
    ohr0 = (iota_r == (idx0 & (_NR - 1))[None, :]).astype(jnp.bfloat16)
    ohr1 = (iota_r == (idx1 & (_NR - 1))[None, :]).astype(jnp.bfloat16)
    ohfull = jnp.concatenate([ohr0, ohr1, ohr0, ohr1], axis=0)
    u_int = lax.dot_general(
        p_ref[...], ohfull, (((1,), (0,)), ((), ())),
        preferred_element_type=jnp.float32)
    parity = lax.broadcasted_iota(jnp.int32, (16, blk), 0) & 1
    q0_b = jnp.broadcast_to((idx0 >> 5)[None, :], (16, blk))
    q1_b = jnp.broadcast_to((idx1 >> 5)[None, :], (16, blk))
    q_int = jnp.where(parity == 1, q1_b, q0_b)
    acc = jnp.zeros((16, blk), jnp.float32)
    for qq in range(_NQ):
        acc = jnp.where(q_int == qq, u_int[qq * 16:(qq + 1) * 16, :], acc)
    zhat_ref[0] = acc

    part = jnp.sum(mu * mu + var - 1.0 - lvc)

    @pl.when(jnp.logical_and(pl.program_id(0) == 0, pl.program_id(1) == 0))
    def _init():
        kl_ref[...] = jnp.zeros_like(kl_ref)

    kl_ref[...] += part


def _codebook_mats(prior_samples):
    ps = prior_samples.astype(jnp.float32)
    ps2 = ps * ps
    j = jnp.arange(16)
    dsel = j // 2
    par = j % 2
    gsel = jnp.arange(2)[:, None, None]
    wa = jnp.where(par[None, None, :] == gsel, ps2[:, dsel][None], 0.0)
    wb = jnp.where(par[None, None, :] == gsel, ps[:, dsel][None], 0.0)
    w_int = jnp.concatenate(
        [wa.reshape(2 * _CB, 16), wb.reshape(2 * _CB, 16)], axis=1)
    w_hi = w_int.astype(jnp.bfloat16)
    wrem = w_int - w_hi.astype(jnp.float32)
    w_mid = wrem.astype(jnp.bfloat16)
    w_lo = (wrem - w_mid.astype(jnp.float32)).astype(jnp.bfloat16)
    w_big = jnp.concatenate(
        [w_hi, w_mid, w_hi, w_mid, w_lo, w_hi], axis=1)

    arr = ps.reshape(_NQ, _NR, _DIM)
    pq = arr[:, :, dsel]
    parr = (jnp.arange(2)[None, :, None, None] == par[None, None, None, :])
    p2 = jnp.where(parr, pq[:, None, :, :], 0.0)
    p_base = p2.transpose(0, 3, 1, 2).reshape(_NQ * 16, 2 * _NR)
    p_hi = p_base.astype(jnp.bfloat16)
    p_lo = (p_base - p_hi.astype(jnp.float32)).astype(jnp.bfloat16)
    p_full = jnp.concatenate([p_hi, p_lo], axis=1)
    return w_big, p_full


def kernel(z, prior_samples):
    batch, chans, hh, ww = z.shape
    spatial = hh * ww
    blk = 4096
    zr = z.reshape(batch, chans, spatial)
    w_big, p_full = _codebook_mats(prior_samples)
    grid = (batch, spatial // blk)
    zhat3, idx3, klsum = pl.pallas_call(
        _body,
        grid=grid,
        in_specs=[
            pl.BlockSpec((1, chans, blk), lambda b, s: (b, 0, s)),
            pl.BlockSpec((2 * _CB, 192), lambda b, s: (0, 0)),
            pl.BlockSpec((_NQ * 16, 128), lambda b, s: (0, 0)),
        ],
        out_specs=[
            pl.BlockSpec((1, 16, blk), lambda b, s: (b, 0, s)),
            pl.BlockSpec((1, 2, blk), lambda b, s: (b, 0, s)),
            pl.BlockSpec((1, 1), lambda b, s: (0, 0)),
        ],
        out_shape=[
            jax.ShapeDtypeStruct((batch, 16, spatial), jnp.float32),
            jax.ShapeDtypeStruct((batch, 2, spatial), jnp.int32),
            jax.ShapeDtypeStruct((1, 1), jnp.float32),
        ],
    )(zr, w_big, p_full)
    zhat = zhat3.reshape(batch, 16, hh, ww)
    indices = idx3.reshape(batch, 2, hh, ww)
    kl_loss = klsum[0, 0] * jnp.float32(_KL_SCALE / (batch * spatial * 2))
    return (zhat, kl_loss, indices)
